# initial kernel scaffold (unmeasured)
import jax
import jax.numpy as jnp
from jax import lax
from jax.experimental import pallas as pl
from jax.experimental.pallas import tpu as pltpu


def kernel(
    x,
):
    def body(*refs):
        pass

    out_shape = jax.ShapeDtypeStruct(..., jnp.float32)
    return pl.pallas_call(body, out_shape=out_shape)(...)


# baseline (device time: 31787 ns/iter reference)
import jax
import jax.numpy as jnp
from jax import lax
from jax.experimental import pallas as pl
from jax.experimental.pallas import tpu as pltpu


def kernel(x):
    m, n = x.shape

    def body(x_ref, out_ref, sbuf, rbuf, send_sems, recv_sems):
        my_x = lax.axis_index("x")
        my_y = lax.axis_index("y")
        y_peer = (my_x, 1 - my_y)
        x_peer = (1 - my_x, my_y)

        barrier_sem = pltpu.get_barrier_semaphore()
        for nbr in (y_peer, x_peer):
            pl.semaphore_signal(
                barrier_sem, inc=1,
                device_id=nbr, device_id_type=pl.DeviceIdType.MESH,
            )
        pl.semaphore_wait(barrier_sem, 2)

        sbuf[...] = x_ref[...].astype(jnp.bfloat16)

        rdma0 = pltpu.make_async_remote_copy(
            src_ref=sbuf,
            dst_ref=rbuf.at[0],
            send_sem=send_sems.at[0],
            recv_sem=recv_sems.at[0],
            device_id=y_peer,
            device_id_type=pl.DeviceIdType.MESH,
        )
        rdma0.start()
        rdma0.wait()
        sbuf[...] = sbuf[...] + rbuf[0]

        rdma1 = pltpu.make_async_remote_copy(
            src_ref=sbuf,
            dst_ref=rbuf.at[1],
            send_sem=send_sems.at[1],
            recv_sem=recv_sems.at[1],
            device_id=x_peer,
            device_id_type=pl.DeviceIdType.MESH,
        )
        rdma1.start()
        rdma1.wait()
        out_ref[...] = (sbuf[...] + rbuf[1]).astype(jnp.float32)

    return pl.pallas_call(
        body,
        out_shape=jax.ShapeDtypeStruct((m, n), jnp.float32),
        in_specs=[pl.BlockSpec(memory_space=pltpu.VMEM)],
        out_specs=pl.BlockSpec(memory_space=pltpu.VMEM),
        scratch_shapes=[
            pltpu.VMEM((m, n), jnp.bfloat16),
            pltpu.VMEM((2, m, n), jnp.bfloat16),
            pltpu.SemaphoreType.DMA((2,)),
            pltpu.SemaphoreType.DMA((2,)),
        ],
        compiler_params=pltpu.CompilerParams(collective_id=0),
    )(x)


# device time: 20442 ns/iter; 1.5550x vs baseline; 1.5550x over previous
import jax
import jax.numpy as jnp
from jax import lax
from jax.experimental import pallas as pl
from jax.experimental.pallas import tpu as pltpu


def kernel(x):
    m, n = x.shape
    half = m // 2

    def body(x_ref, out_ref, sbuf, rbuf, send_sems, recv_sems):
        my_x = lax.axis_index("x")
        my_y = lax.axis_index("y")
        y_peer = (my_x, 1 - my_y)
        x_peer = (1 - my_x, my_y)

        barrier_sem = pltpu.get_barrier_semaphore()
        for nbr in (y_peer, x_peer):
            pl.semaphore_signal(
                barrier_sem, inc=1,
                device_id=nbr, device_id_type=pl.DeviceIdType.MESH,
            )
        pl.semaphore_wait(barrier_sem, 2)

        a = pl.ds(0, half)
        b = pl.ds(half, half)

        def exchange(src, dst, sem_idx, peer):
            return pltpu.make_async_remote_copy(
                src_ref=src,
                dst_ref=dst,
                send_sem=send_sems.at[sem_idx],
                recv_sem=recv_sems.at[sem_idx],
                device_id=peer,
                device_id_type=pl.DeviceIdType.MESH,
            )

        sbuf[a, :] = x_ref[a, :].astype(jnp.bfloat16)
        rdma_a0 = exchange(sbuf.at[a, :], rbuf.at[0, a, :], 0, y_peer)
        rdma_a0.start()
        sbuf[b, :] = x_ref[b, :].astype(jnp.bfloat16)
        rdma_b0 = exchange(sbuf.at[b, :], rbuf.at[0, b, :], 1, x_peer)
        rdma_b0.start()

        rdma_a0.wait()
        sbuf[a, :] = sbuf[a, :] + rbuf[0, a, :]
        rdma_a1 = exchange(sbuf.at[a, :], rbuf.at[1, a, :], 2, x_peer)
        rdma_a1.start()
        rdma_b0.wait()
        sbuf[b, :] = sbuf[b, :] + rbuf[0, b, :]
        rdma_b1 = exchange(sbuf.at[b, :], rbuf.at[1, b, :], 3, y_peer)
        rdma_b1.start()

        rdma_a1.wait()
        out_ref[a, :] = (sbuf[a, :] + rbuf[1, a, :]).astype(jnp.float32)
        rdma_b1.wait()
        out_ref[b, :] = (sbuf[b, :] + rbuf[1, b, :]).astype(jnp.float32)

    return pl.pallas_call(
        body,
        out_shape=jax.ShapeDtypeStruct((m, n), jnp.float32),
        in_specs=[pl.BlockSpec(memory_space=pltpu.VMEM)],
        out_specs=pl.BlockSpec(memory_space=pltpu.VMEM),
        scratch_shapes=[
            pltpu.VMEM((m, n), jnp.bfloat16),
            pltpu.VMEM((2, m, n), jnp.bfloat16),
            pltpu.SemaphoreType.DMA((4,)),
            pltpu.SemaphoreType.DMA((4,)),
        ],
        compiler_params=pltpu.CompilerParams(collective_id=0),
    )(x)


# device time: 19199 ns/iter; 1.6557x vs baseline; 1.0647x over previous
import jax
import jax.numpy as jnp
from jax import lax
from jax.experimental import pallas as pl
from jax.experimental.pallas import tpu as pltpu

NQ = 4
ORDER = (0, 2, 1, 3)


def kernel(x):
    m, n = x.shape
    q = m // NQ

    def body(x_ref, out_ref, sbuf, rbuf, send_sems, recv_sems):
        my_x = lax.axis_index("x")
        my_y = lax.axis_index("y")
        y_peer = (my_x, 1 - my_y)
        x_peer = (1 - my_x, my_y)
        first_peer = {0: y_peer, 1: y_peer, 2: x_peer, 3: x_peer}
        second_peer = {0: x_peer, 1: x_peer, 2: y_peer, 3: y_peer}

        barrier_sem = pltpu.get_barrier_semaphore()
        for nbr in (y_peer, x_peer):
            pl.semaphore_signal(
                barrier_sem, inc=1,
                device_id=nbr, device_id_type=pl.DeviceIdType.MESH,
            )
        pl.semaphore_wait(barrier_sem, 2)

        def exchange(src, dst, sem_idx, peer):
            return pltpu.make_async_remote_copy(
                src_ref=src,
                dst_ref=dst,
                send_sem=send_sems.at[sem_idx],
                recv_sem=recv_sems.at[sem_idx],
                device_id=peer,
                device_id_type=pl.DeviceIdType.MESH,
            )

        sl = {i: pl.ds(i * q, q) for i in range(NQ)}

        rd1 = {}
        for i in ORDER:
            sbuf[sl[i], :] = x_ref[sl[i], :].astype(jnp.bfloat16)
            rd1[i] = exchange(
                sbuf.at[sl[i], :], rbuf.at[0, sl[i], :], i, first_peer[i]
            )
            rd1[i].start()

        rd2 = {}
        for i in ORDER:
            rd1[i].wait()
            sbuf[sl[i], :] = sbuf[sl[i], :] + rbuf[0, sl[i], :]
            rd2[i] = exchange(
                sbuf.at[sl[i], :], rbuf.at[1, sl[i], :], NQ + i, second_peer[i]
            )
            rd2[i].start()

        for i in ORDER:
            rd2[i].wait()
            out_ref[sl[i], :] = (
                sbuf[sl[i], :] + rbuf[1, sl[i], :]
            ).astype(jnp.float32)

    return pl.pallas_call(
        body,
        out_shape=jax.ShapeDtypeStruct((m, n), jnp.float32),
        in_specs=[pl.BlockSpec(memory_space=pltpu.VMEM)],
        out_specs=pl.BlockSpec(memory_space=pltpu.VMEM),
        scratch_shapes=[
            pltpu.VMEM((m, n), jnp.bfloat16),
            pltpu.VMEM((2, m, n), jnp.bfloat16),
            pltpu.SemaphoreType.DMA((2 * NQ,)),
            pltpu.SemaphoreType.DMA((2 * NQ,)),
        ],
        compiler_params=pltpu.CompilerParams(collective_id=0),
    )(x)
